# baseline (speedup 1.0000x reference)
import functools

import jax
import jax.numpy as jnp
from jax import lax
from jax.experimental import pallas as pl
from jax.experimental.pallas import tpu as pltpu
from jax.experimental.pallas import tpu_sc as plsc

N_NODES = 10000
N_EDGES = 320000
D_IN = 128
D_HID = 256
D_OUT = 128

ROW_BLOCK = 1000

LANES = 16
NW = 32
EW = N_EDGES // NW
CHUNK = 128
NFULL = EW // CHUNK
TAIL = EW - NFULL * CHUNK
NGROUPS = CHUNK // LANES
D_PACK = D_OUT // 2


def _encode_body(x_ref, w1_ref, b1_ref, w2_ref, b2_ref, z_ref):
    h = jnp.dot(x_ref[...], w1_ref[...], preferred_element_type=jnp.float32)
    h = jnp.maximum(h + b1_ref[...], 0.0)
    z = jnp.dot(h, w2_ref[...], preferred_element_type=jnp.float32)
    z_ref[...] = (z + b2_ref[...]).astype(jnp.bfloat16)


def _encode(x, W1, b1, W2, b2):
    n = x.shape[0]
    grid = (n // ROW_BLOCK,)
    return pl.pallas_call(
        _encode_body,
        grid=grid,
        in_specs=[
            pl.BlockSpec((ROW_BLOCK, D_IN), lambda i: (i, 0)),
            pl.BlockSpec((D_IN, D_HID), lambda i: (0, 0)),
            pl.BlockSpec((1, D_HID), lambda i: (0, 0)),
            pl.BlockSpec((D_HID, D_OUT), lambda i: (0, 0)),
            pl.BlockSpec((1, D_OUT), lambda i: (0, 0)),
        ],
        out_specs=pl.BlockSpec((ROW_BLOCK, D_OUT), lambda i: (i, 0)),
        out_shape=jax.ShapeDtypeStruct((n, D_OUT), jnp.bfloat16),
    )(x, W1, b1.reshape(1, -1), W2, b2.reshape(1, -1))


def _decode_body(z_hbm, ei_hbm, out_hbm,
                 sidx, didx, sr0, dr0, sr1, dr1, outb, tmat, sem0, sem1):
    cid = lax.axis_index("c")
    sid = lax.axis_index("s")
    wid = sid * 2 + cid
    ebase = wid * EW
    rowflat = jnp.arange(LANES, dtype=jnp.int32) * LANES

    pltpu.sync_copy(ei_hbm.at[pl.ds(ebase, EW)], sidx)
    pltpu.sync_copy(ei_hbm.at[pl.ds(N_EDGES + ebase, EW)], didx)

    srs = (sr0, sr1)
    drs = (dr0, dr1)
    sems = (sem0, sem1)

    def issue(slot, c):
        off = c * CHUNK
        pltpu.async_copy(z_hbm.at[sidx.at[pl.ds(off, CHUNK)]],
                         srs[slot], sems[slot])
        pltpu.async_copy(z_hbm.at[didx.at[pl.ds(off, CHUNK)]],
                         drs[slot], sems[slot])

    def drain(slot):
        pltpu.make_async_copy(z_hbm.at[sidx.at[pl.ds(0, CHUNK)]],
                              srs[slot], sems[slot]).wait()
        pltpu.make_async_copy(z_hbm.at[didx.at[pl.ds(0, CHUNK)]],
                              drs[slot], sems[slot]).wait()

    def compute(slot, c):
        sr, dr = srs[slot], drs[slot]
        off = c * CHUNK

        @plsc.parallel_loop(0, NGROUPS, unroll=4)
        def group_body(g):
            e0 = g * LANES
            t0 = g * LANES * LANES
            for j in range(LANES):
                e = e0 + j
                accb = None
                for k in range(D_PACK // LANES):
                    s = plsc.bitcast(sr[e, pl.ds(k * LANES, LANES)],
                                     jnp.bfloat16)
                    d = plsc.bitcast(dr[e, pl.ds(k * LANES, LANES)],
                                     jnp.bfloat16)
                    p = s * d
                    accb = p if accb is None else accb + p
                a, b = plsc.unpack(accb, format=plsc.PackFormat.INTERLEAVED)
                tmat[pl.ds(t0 + j * LANES, LANES)] = a + b
            out = plsc.load_gather(tmat, [t0 + rowflat])
            for lane in range(1, LANES):
                out = out + plsc.load_gather(tmat, [t0 + rowflat + lane])
            outb[pl.ds(off + e0, LANES)] = out

    issue(0, 0)

    def pair_body(p, carry):
        issue(1, 2 * p + 1)
        drain(0)
        compute(0, 2 * p)

        @pl.when(p < NFULL // 2 - 1)
        def _():
            issue(0, 2 * p + 2)

        drain(1)
        compute(1, 2 * p + 1)
        return carry

    lax.fori_loop(0, NFULL // 2, pair_body, 0)

    toff = NFULL * CHUNK
    cps = pltpu.async_copy(z_hbm.at[sidx.at[pl.ds(toff, TAIL)]],
                           sr0.at[pl.ds(0, TAIL)], sem0)
    cpd = pltpu.async_copy(z_hbm.at[didx.at[pl.ds(toff, TAIL)]],
                           dr0.at[pl.ds(0, TAIL)], sem0)
    cps.wait()
    cpd.wait()
    for j in range(TAIL):
        accb = None
        for k in range(D_PACK // LANES):
            s = plsc.bitcast(sr0[j, pl.ds(k * LANES, LANES)], jnp.bfloat16)
            d = plsc.bitcast(dr0[j, pl.ds(k * LANES, LANES)], jnp.bfloat16)
            p = s * d
            accb = p if accb is None else accb + p
        a, b = plsc.unpack(accb, format=plsc.PackFormat.INTERLEAVED)
        tmat[pl.ds(j * LANES, LANES)] = a + b
    out = plsc.load_gather(tmat, [rowflat])
    for lane in range(1, LANES):
        out = out + plsc.load_gather(tmat, [rowflat + lane])
    outb[pl.ds(toff, LANES)] = out

    pltpu.sync_copy(outb, out_hbm.at[pl.ds(ebase, EW)])


def _decode(z, edge_index):
    mesh = plsc.VectorSubcoreMesh(core_axis_name="c", subcore_axis_name="s")
    k = functools.partial(
        pl.kernel,
        mesh=mesh,
        out_type=jax.ShapeDtypeStruct((N_EDGES,), jnp.float32),
        scratch_types=[
            pltpu.VMEM((EW,), jnp.int32),
            pltpu.VMEM((EW,), jnp.int32),
            pltpu.VMEM((CHUNK, D_PACK), jnp.float32),
            pltpu.VMEM((CHUNK, D_PACK), jnp.float32),
            pltpu.VMEM((CHUNK, D_PACK), jnp.float32),
            pltpu.VMEM((CHUNK, D_PACK), jnp.float32),
            pltpu.VMEM((EW,), jnp.float32),
            pltpu.VMEM((NGROUPS * LANES * LANES,), jnp.float32),
            pltpu.SemaphoreType.DMA,
            pltpu.SemaphoreType.DMA,
        ],
        compiler_params=pltpu.CompilerParams(needs_layout_passes=False, disable_bounds_checks=True, use_tc_tiling_on_sc=False),
    )(_decode_body)
    return k(z, edge_index.reshape(-1))


def kernel(x, edge_index, W1, b1, W2, b2):
    zb = _encode(x, W1, b1, W2, b2)
    zp = jax.lax.bitcast_convert_type(
        zb.reshape(N_NODES, D_PACK, 2), jnp.float32)
    return _decode(zp, edge_index)


# speedup vs baseline: 1.3389x; 1.3389x over previous
import functools

import jax
import jax.numpy as jnp
from jax import lax
from jax.experimental import pallas as pl
from jax.experimental.pallas import tpu as pltpu
from jax.experimental.pallas import tpu_sc as plsc

N_NODES = 10000
N_EDGES = 320000
D_IN = 128
D_HID = 256
D_OUT = 128

ROW_BLOCK = 1000

LANES = 16
NW = 32
EW = N_EDGES // NW
CHUNK = 128
NFULL = EW // CHUNK
TAIL = EW - NFULL * CHUNK
NGROUPS = CHUNK // LANES
D_PACK = D_OUT // 2
PITCH = 17


def _encode_body(x_ref, w1_ref, b1_ref, w2_ref, b2_ref, z_ref):
    h = jnp.dot(x_ref[...], w1_ref[...], preferred_element_type=jnp.float32)
    h = jnp.maximum(h + b1_ref[...], 0.0)
    z = jnp.dot(h, w2_ref[...], preferred_element_type=jnp.float32)
    z_ref[...] = (z + b2_ref[...]).astype(jnp.bfloat16)


def _encode(x, W1, b1, W2, b2):
    n = x.shape[0]
    grid = (n // ROW_BLOCK,)
    return pl.pallas_call(
        _encode_body,
        grid=grid,
        in_specs=[
            pl.BlockSpec((ROW_BLOCK, D_IN), lambda i: (i, 0)),
            pl.BlockSpec((D_IN, D_HID), lambda i: (0, 0)),
            pl.BlockSpec((1, D_HID), lambda i: (0, 0)),
            pl.BlockSpec((D_HID, D_OUT), lambda i: (0, 0)),
            pl.BlockSpec((1, D_OUT), lambda i: (0, 0)),
        ],
        out_specs=pl.BlockSpec((ROW_BLOCK, D_OUT), lambda i: (i, 0)),
        out_shape=jax.ShapeDtypeStruct((n, D_OUT), jnp.bfloat16),
    )(x, W1, b1.reshape(1, -1), W2, b2.reshape(1, -1))


def _decode_body(z_hbm, ei_hbm, out_hbm,
                 sidx, didx, sr0, dr0, sr1, dr1, outb, tmat, sem0, sem1):
    cid = lax.axis_index("c")
    sid = lax.axis_index("s")
    wid = sid * 2 + cid
    ebase = wid * EW
    colflat = jnp.arange(LANES, dtype=jnp.int32) * PITCH

    pltpu.sync_copy(ei_hbm.at[pl.ds(ebase, EW)], sidx)
    pltpu.sync_copy(ei_hbm.at[pl.ds(N_EDGES + ebase, EW)], didx)

    srs = (sr0, sr1)
    drs = (dr0, dr1)
    sems = (sem0, sem1)

    def issue(slot, c):
        off = c * CHUNK
        pltpu.async_copy(z_hbm.at[sidx.at[pl.ds(off, CHUNK)]],
                         srs[slot], sems[slot])
        pltpu.async_copy(z_hbm.at[didx.at[pl.ds(off, CHUNK)]],
                         drs[slot], sems[slot])

    def drain(slot):
        pltpu.make_async_copy(z_hbm.at[sidx.at[pl.ds(0, CHUNK)]],
                              srs[slot], sems[slot]).wait()
        pltpu.make_async_copy(z_hbm.at[didx.at[pl.ds(0, CHUNK)]],
                              drs[slot], sems[slot]).wait()

    def compute(slot, c):
        sr, dr = srs[slot], drs[slot]
        off = c * CHUNK

        @plsc.parallel_loop(0, NGROUPS, unroll=2)
        def group_body(g):
            e0 = g * LANES
            t0 = g * LANES * PITCH
            for j in range(LANES):
                e = e0 + j
                accb = None
                for k in range(D_PACK // LANES):
                    s = plsc.bitcast(sr[e, pl.ds(k * LANES, LANES)],
                                     jnp.bfloat16)
                    d = plsc.bitcast(dr[e, pl.ds(k * LANES, LANES)],
                                     jnp.bfloat16)
                    p = s * d
                    accb = p if accb is None else accb + p
                a, b = plsc.unpack(accb, format=plsc.PackFormat.INTERLEAVED)
                tmat[pl.ds(t0 + j * PITCH, LANES)] = a + b
            out = plsc.load_gather(tmat, [t0 + colflat])
            for c in range(1, LANES):
                out = out + plsc.load_gather(tmat, [t0 + colflat + c])
            outb[pl.ds(off + e0, LANES)] = out

    issue(0, 0)

    def pair_body(p, carry):
        issue(1, 2 * p + 1)
        drain(0)
        compute(0, 2 * p)

        @pl.when(p < NFULL // 2 - 1)
        def _():
            issue(0, 2 * p + 2)

        drain(1)
        compute(1, 2 * p + 1)
        return carry

    lax.fori_loop(0, NFULL // 2, pair_body, 0)

    toff = NFULL * CHUNK
    cps = pltpu.async_copy(z_hbm.at[sidx.at[pl.ds(toff, TAIL)]],
                           sr0.at[pl.ds(0, TAIL)], sem0)
    cpd = pltpu.async_copy(z_hbm.at[didx.at[pl.ds(toff, TAIL)]],
                           dr0.at[pl.ds(0, TAIL)], sem0)
    cps.wait()
    cpd.wait()
    for j in range(TAIL):
        accb = None
        for k in range(D_PACK // LANES):
            s = plsc.bitcast(sr0[j, pl.ds(k * LANES, LANES)], jnp.bfloat16)
            d = plsc.bitcast(dr0[j, pl.ds(k * LANES, LANES)], jnp.bfloat16)
            p = s * d
            accb = p if accb is None else accb + p
        a, b = plsc.unpack(accb, format=plsc.PackFormat.INTERLEAVED)
        tmat[pl.ds(j * PITCH, LANES)] = a + b
    out = plsc.load_gather(tmat, [colflat])
    for c in range(1, LANES):
        out = out + plsc.load_gather(tmat, [colflat + c])
    outb[pl.ds(toff, LANES)] = out

    pltpu.sync_copy(outb, out_hbm.at[pl.ds(ebase, EW)])


def _decode(z, edge_index):
    mesh = plsc.VectorSubcoreMesh(core_axis_name="c", subcore_axis_name="s")
    k = functools.partial(
        pl.kernel,
        mesh=mesh,
        out_type=jax.ShapeDtypeStruct((N_EDGES,), jnp.float32),
        scratch_types=[
            pltpu.VMEM((EW,), jnp.int32),
            pltpu.VMEM((EW,), jnp.int32),
            pltpu.VMEM((CHUNK, D_PACK), jnp.float32),
            pltpu.VMEM((CHUNK, D_PACK), jnp.float32),
            pltpu.VMEM((CHUNK, D_PACK), jnp.float32),
            pltpu.VMEM((CHUNK, D_PACK), jnp.float32),
            pltpu.VMEM((EW,), jnp.float32),
            pltpu.VMEM((NGROUPS * LANES * PITCH,), jnp.float32),
            pltpu.SemaphoreType.DMA,
            pltpu.SemaphoreType.DMA,
        ],
        compiler_params=pltpu.CompilerParams(needs_layout_passes=False, disable_bounds_checks=True, use_tc_tiling_on_sc=False),
    )(_decode_body)
    return k(z, edge_index.reshape(-1))


def kernel(x, edge_index, W1, b1, W2, b2):
    zb = _encode(x, W1, b1, W2, b2)
    zp = jax.lax.bitcast_convert_type(
        zb.reshape(N_NODES, D_PACK, 2), jnp.float32)
    return _decode(zp, edge_index)
